# Initial kernel scaffold; baseline (speedup 1.0000x reference)
import jax
import jax.numpy as jnp
from jax.experimental import pallas as pl


def kernel(inputs, tables):
    raise NotImplementedError("write your pallas kernel here")


# speedup vs baseline: 5.6082x; 5.6082x over previous
import functools

import jax
import jax.numpy as jnp
from jax import lax
from jax.experimental import pallas as pl
from jax.experimental.pallas import tpu as pltpu
from jax.experimental.pallas import tpu_sc as plsc

_N_COLS = 26
_VOCAB = 1000
_D = 64
_PER = 208
_CHUNK = 128


def _make_sc_gather(n_total):
    info = plsc.get_sparse_core_info()
    nc, ns = info.num_cores, info.num_subcores
    nw = nc * ns
    n_per_w = n_total // nw
    assert n_total % nw == 0 and n_per_w % _PER == 0 and n_per_w % _CHUNK == 0
    n_chunks = n_per_w // _CHUNK

    mesh = plsc.VectorSubcoreMesh(core_axis_name="c", subcore_axis_name="s")

    @functools.partial(
        pl.kernel,
        mesh=mesh,
        compiler_params=pltpu.CompilerParams(use_tc_tiling_on_sc=False),
        out_type=jax.ShapeDtypeStruct((n_total, _D), jnp.float32),
        scratch_types=[
            pltpu.VMEM((n_per_w,), jnp.int32),
            pltpu.VMEM((_PER,), jnp.int32),
            pltpu.VMEM((_CHUNK, _D), jnp.float32),
            pltpu.SemaphoreType.DMA,
        ],
    )
    def k(idx_hbm, tab_hbm, off_hbm, out_hbm, gidx_v, off_v, rows_v, gsem):
        wid = lax.axis_index("s") * nc + lax.axis_index("c")
        base = wid * n_per_w
        pltpu.sync_copy(idx_hbm.at[pl.ds(base, n_per_w)], gidx_v)
        pltpu.sync_copy(off_hbm, off_v)

        def add_offsets(g, carry):
            s0 = g * _PER
            for kk in range(_PER // 16):
                s = s0 + kk * 16
                gidx_v[pl.ds(s, 16)] = (
                    gidx_v[pl.ds(s, 16)] + off_v[pl.ds(kk * 16, 16)]
                )
            return carry

        lax.fori_loop(0, n_per_w // _PER, add_offsets, 0)

        def gather_chunk(c, carry):
            pltpu.async_copy(
                tab_hbm.at[gidx_v.at[pl.ds(c * _CHUNK, _CHUNK)]],
                rows_v,
                gsem,
            ).wait()
            pltpu.sync_copy(rows_v, out_hbm.at[pl.ds(base + c * _CHUNK, _CHUNK)])
            return carry

        lax.fori_loop(0, n_chunks, gather_chunk, 0)

    return k


def kernel(inputs, tables):
    b, l, n_cols = inputs.shape
    _, vocab, d = tables.shape
    n_total = b * l * n_cols
    idx_flat = inputs.astype(jnp.int32).reshape(n_total)
    tab_flat = tables.reshape(n_cols * vocab, d)
    offsets = (jnp.arange(_PER, dtype=jnp.int32) % n_cols) * vocab
    out_flat = _make_sc_gather(n_total)(idx_flat, tab_flat, offsets)
    return out_flat.reshape(b, l, n_cols * d)


# speedup vs baseline: 6.8341x; 1.2186x over previous
import functools

import jax
import jax.numpy as jnp
from jax import lax
from jax.experimental import pallas as pl
from jax.experimental.pallas import tpu as pltpu
from jax.experimental.pallas import tpu_sc as plsc

_N_COLS = 26
_VOCAB = 1000
_D = 64
_PER = 208
_CHUNK = 128
_K = 5


def _make_sc_gather(n_total):
    info = plsc.get_sparse_core_info()
    nc, ns = info.num_cores, info.num_subcores
    nw = nc * ns
    n_per_w = n_total // nw
    assert n_total % nw == 0 and n_per_w % _PER == 0
    n_chunks = n_per_w // _CHUNK
    n_groups = n_chunks // _K
    assert n_chunks % _K == 0 and n_groups % 2 == 0

    mesh = plsc.VectorSubcoreMesh(core_axis_name="c", subcore_axis_name="s")

    @functools.partial(
        pl.kernel,
        mesh=mesh,
        compiler_params=pltpu.CompilerParams(use_tc_tiling_on_sc=False),
        out_type=jax.ShapeDtypeStruct((n_total, _D), jnp.float32),
        scratch_types=[
            pltpu.VMEM((n_per_w,), jnp.int32),
            pltpu.VMEM((_PER,), jnp.int32),
            pltpu.VMEM((2, _K, _CHUNK, _D), jnp.float32),
            pltpu.SemaphoreType.DMA,
            pltpu.SemaphoreType.DMA,
            pltpu.SemaphoreType.DMA,
            pltpu.SemaphoreType.DMA,
        ],
    )
    def k(idx_hbm, tab_hbm, off_hbm, out_hbm, gidx_v, off_v, bufs, gs0, gs1,
          os0, os1):
        wid = lax.axis_index("s") * nc + lax.axis_index("c")
        base = wid * n_per_w
        pltpu.sync_copy(idx_hbm.at[pl.ds(base, n_per_w)], gidx_v)
        pltpu.sync_copy(off_hbm, off_v)

        def add_offsets(g, carry):
            s0 = g * _PER
            for kk in range(_PER // 16):
                s = s0 + kk * 16
                gidx_v[pl.ds(s, 16)] = (
                    gidx_v[pl.ds(s, 16)] + off_v[pl.ds(kk * 16, 16)]
                )
            return carry

        lax.fori_loop(0, n_per_w // _PER, add_offsets, 0)

        gsems = (gs0, gs1)
        osems = (os0, os1)

        def fire_gathers(g, half):
            c0 = g * (_K * _CHUNK)
            for b in range(_K):
                pltpu.async_copy(
                    tab_hbm.at[gidx_v.at[pl.ds(c0 + b * _CHUNK, _CHUNK)]],
                    bufs.at[half, b],
                    gsems[half],
                )

        def drain_gathers(half):
            for b in range(_K):
                pltpu.make_async_copy(
                    out_hbm.at[pl.ds(base, _CHUNK)],
                    bufs.at[half, b],
                    gsems[half],
                ).wait()

        def fire_outs(g, half):
            c0 = g * (_K * _CHUNK)
            for b in range(_K):
                pltpu.async_copy(
                    bufs.at[half, b],
                    out_hbm.at[pl.ds(base + c0 + b * _CHUNK, _CHUNK)],
                    osems[half],
                )

        def drain_outs(half):
            for b in range(_K):
                pltpu.make_async_copy(
                    bufs.at[half, b],
                    out_hbm.at[pl.ds(base, _CHUNK)],
                    osems[half],
                ).wait()

        fire_gathers(0, 0)
        fire_gathers(1, 1)

        def body(gp, carry):
            g0 = 2 * gp
            g1 = g0 + 1
            drain_gathers(0)
            fire_outs(g0, 0)
            drain_gathers(1)
            fire_outs(g1, 1)
            drain_outs(0)

            @pl.when(g0 + 2 < n_groups)
            def _():
                fire_gathers(g0 + 2, 0)

            drain_outs(1)

            @pl.when(g1 + 2 < n_groups)
            def _():
                fire_gathers(g1 + 2, 1)

            return carry

        lax.fori_loop(0, n_groups // 2, body, 0)

    return k


def kernel(inputs, tables):
    b, l, n_cols = inputs.shape
    _, vocab, d = tables.shape
    n_total = b * l * n_cols
    idx_flat = inputs.astype(jnp.int32).reshape(n_total)
    tab_flat = tables.reshape(n_cols * vocab, d)
    offsets = (jnp.arange(_PER, dtype=jnp.int32) % n_cols) * vocab
    out_flat = _make_sc_gather(n_total)(idx_flat, tab_flat, offsets)
    return out_flat.reshape(b, l, n_cols * d)


# speedup vs baseline: 9.3884x; 1.3738x over previous
import functools

import jax
import jax.numpy as jnp
from jax import lax
from jax.experimental import pallas as pl
from jax.experimental.pallas import tpu as pltpu
from jax.experimental.pallas import tpu_sc as plsc
from jax.experimental import layout as _jlayout

_N_COLS = 26
_VOCAB = 1000
_D = 64
_L = 20
_B = 1024
_PB = _L * _N_COLS
_SR = 8 * _N_COLS
_GC = _SR // 2


def _make_sc_gather():
    info = plsc.get_sparse_core_info()
    nc, ns = info.num_cores, info.num_subcores
    nw = nc * ns
    nb_per_w = _B // nw
    nblk = nb_per_w // 8
    n_per_w = nb_per_w * _PB
    n_stripes = nblk * _L
    n_bb = _B // 8
    assert _B % (8 * nw) == 0 and n_stripes % 2 == 0

    mesh = plsc.VectorSubcoreMesh(core_axis_name="c", subcore_axis_name="s")

    @functools.partial(
        pl.kernel,
        mesh=mesh,
        compiler_params=pltpu.CompilerParams(
            use_tc_tiling_on_sc=False, needs_layout_passes=False
        ),
        out_type=jax.ShapeDtypeStruct((_B * _PB, _D), jnp.float32),
        scratch_types=[
            pltpu.VMEM((n_per_w,), jnp.int32),
            pltpu.VMEM((_SR,), jnp.int32),
            pltpu.VMEM((_SR,), jnp.int32),
            pltpu.VMEM((2, _SR), jnp.int32),
            pltpu.VMEM((2, _SR, _D), jnp.float32),
            pltpu.SemaphoreType.DMA,
            pltpu.SemaphoreType.DMA,
            pltpu.SemaphoreType.DMA,
            pltpu.SemaphoreType.DMA,
        ],
    )
    def k(idx_hbm, tab_hbm, perm_hbm, offp_hbm, out_hbm, idx_v, perm_v,
          offp_v, gidx_v, bufs, gs0, gs1, os0, os1):
        wid = lax.axis_index("s") * nc + lax.axis_index("c")
        base = wid * n_per_w
        pltpu.sync_copy(idx_hbm.at[pl.ds(base, n_per_w)], idx_v)
        pltpu.sync_copy(perm_hbm, perm_v)
        pltpu.sync_copy(offp_hbm, offp_v)

        gsems = (gs0, gs1)
        osems = (os0, os1)

        def build_gidx(s, half):
            blk = s // _L
            l = s % _L
            sbase = blk * (8 * _PB) + l * _N_COLS
            for kk in range(_SR // 16):
                sl = pl.ds(kk * 16, 16)
                src = perm_v[sl] + sbase
                vals = plsc.load_gather(idx_v, [src])
                gidx_v[half, sl] = vals + offp_v[sl]

        def fire_gathers(half):
            for j in range(2):
                pltpu.async_copy(
                    tab_hbm.at[gidx_v.at[half, pl.ds(j * _GC, _GC)]],
                    bufs.at[half, pl.ds(j * _GC, _GC)],
                    gsems[half],
                )

        def drain_gathers(half):
            for j in range(2):
                pltpu.make_async_copy(
                    tab_hbm.at[pl.ds(0, _GC)],
                    bufs.at[half, pl.ds(j * _GC, _GC)],
                    gsems[half],
                ).wait()

        def fire_out(s, half):
            blk = s // _L
            l = s % _L
            row0 = (l * _B + wid * nb_per_w + 8 * blk) * _N_COLS
            pltpu.async_copy(
                bufs.at[half],
                out_hbm.at[pl.ds(row0, _SR)],
                osems[half],
            )

        def drain_out(half):
            pltpu.make_async_copy(
                bufs.at[half],
                out_hbm.at[pl.ds(0, _SR)],
                osems[half],
            ).wait()

        build_gidx(0, 0)
        fire_gathers(0)
        build_gidx(1, 1)
        fire_gathers(1)

        def body(sp, carry):
            s0 = 2 * sp
            s1 = s0 + 1
            drain_gathers(0)
            fire_out(s0, 0)
            drain_gathers(1)
            fire_out(s1, 1)

            @pl.when(s0 + 2 < n_stripes)
            def _():
                build_gidx(s0 + 2, 0)
                drain_out(0)
                fire_gathers(0)

            @pl.when(s1 + 2 < n_stripes)
            def _():
                build_gidx(s1 + 2, 1)
                drain_out(1)
                fire_gathers(1)

            @pl.when(s0 + 2 >= n_stripes)
            def _():
                drain_out(0)

            @pl.when(s1 + 2 >= n_stripes)
            def _():
                drain_out(1)

            return carry

        lax.fori_loop(0, n_stripes // 2, body, 0)

    return k


def kernel(inputs, tables):
    b, l, n_cols = inputs.shape
    _, vocab, d = tables.shape
    idx_flat = inputs.astype(jnp.int32).reshape(b * l * n_cols)
    tab_flat = tables.reshape(n_cols * vocab, d)
    j = jnp.arange(_SR, dtype=jnp.int32)
    col = j % _N_COLS
    perm = (j // _N_COLS) * _PB + col
    offp = col * vocab
    out2d = _make_sc_gather()(idx_flat, tab_flat, perm, offp)
    return out2d.reshape(_L, _B, _N_COLS * _D).transpose(1, 0, 2)


# speedup vs baseline: 17.3758x; 1.8508x over previous
import functools

import jax
import jax.numpy as jnp
from jax import lax
from jax.experimental import pallas as pl
from jax.experimental.pallas import tpu as pltpu
from jax.experimental.pallas import tpu_sc as plsc
from jax.experimental import layout as _jlayout

_N_COLS = 26
_VOCAB = 1000
_D = 64
_L = 20
_B = 1024
_PB = _L * _N_COLS
_SR = 8 * _N_COLS
_GC = _SR // 2


def _make_sc_gather(lp, l0):
    info = plsc.get_sparse_core_info()
    nc, ns = info.num_cores, info.num_subcores
    nw = nc * ns
    nb_per_w = _B // nw
    nblk = nb_per_w // 8
    n_per_w = nb_per_w * _PB
    n_stripes = nblk * lp
    assert _B % (8 * nw) == 0 and n_stripes % 2 == 0

    mesh = plsc.VectorSubcoreMesh(core_axis_name="c", subcore_axis_name="s")

    @functools.partial(
        pl.kernel,
        mesh=mesh,
        compiler_params=pltpu.CompilerParams(
            use_tc_tiling_on_sc=False, needs_layout_passes=False
        ),
        out_type=jax.ShapeDtypeStruct((lp * _B * _N_COLS, _D), jnp.float32),
        scratch_types=[
            pltpu.VMEM((n_per_w,), jnp.int32),
            pltpu.VMEM((_SR,), jnp.int32),
            pltpu.VMEM((_SR,), jnp.int32),
            pltpu.VMEM((2, _SR), jnp.int32),
            pltpu.VMEM((2, _SR, _D), jnp.float32),
            pltpu.SemaphoreType.DMA,
            pltpu.SemaphoreType.DMA,
            pltpu.SemaphoreType.DMA,
            pltpu.SemaphoreType.DMA,
        ],
    )
    def k(idx_hbm, tab_hbm, perm_hbm, offp_hbm, out_hbm, idx_v, perm_v,
          offp_v, gidx_v, bufs, gs0, gs1, os0, os1):
        wid = lax.axis_index("s") * nc + lax.axis_index("c")
        base = wid * n_per_w
        pltpu.sync_copy(idx_hbm.at[pl.ds(base, n_per_w)], idx_v)
        pltpu.sync_copy(perm_hbm, perm_v)
        pltpu.sync_copy(offp_hbm, offp_v)

        gsems = (gs0, gs1)
        osems = (os0, os1)

        def build_gidx(s, half):
            blk = s // lp
            l = l0 + s % lp
            sbase = blk * (8 * _PB) + l * _N_COLS
            for kk in range(_SR // 16):
                sl = pl.ds(kk * 16, 16)
                src = perm_v[sl] + sbase
                vals = plsc.load_gather(idx_v, [src])
                gidx_v[half, sl] = vals + offp_v[sl]

        def fire_gathers(half):
            for j in range(2):
                pltpu.async_copy(
                    tab_hbm.at[gidx_v.at[half, pl.ds(j * _GC, _GC)]],
                    bufs.at[half, pl.ds(j * _GC, _GC)],
                    gsems[half],
                )

        def drain_gathers(half):
            for j in range(2):
                pltpu.make_async_copy(
                    tab_hbm.at[pl.ds(0, _GC)],
                    bufs.at[half, pl.ds(j * _GC, _GC)],
                    gsems[half],
                ).wait()

        def fire_out(s, half):
            blk = s // lp
            lrel = s % lp
            row0 = (lrel * (_B // 8) + wid * nblk + blk) * _SR
            pltpu.async_copy(
                bufs.at[half],
                out_hbm.at[pl.ds(row0, _SR)],
                osems[half],
            )

        def drain_out(half):
            pltpu.make_async_copy(
                bufs.at[half],
                out_hbm.at[pl.ds(0, _SR)],
                osems[half],
            ).wait()

        build_gidx(0, 0)
        fire_gathers(0)
        build_gidx(1, 1)
        fire_gathers(1)

        def body(sp, carry):
            s0 = 2 * sp
            s1 = s0 + 1
            drain_gathers(0)
            fire_out(s0, 0)
            drain_gathers(1)
            fire_out(s1, 1)

            @pl.when(s0 + 2 < n_stripes)
            def _():
                build_gidx(s0 + 2, 0)
                drain_out(0)
                fire_gathers(0)

            @pl.when(s1 + 2 < n_stripes)
            def _():
                build_gidx(s1 + 2, 1)
                drain_out(1)
                fire_gathers(1)

            @pl.when(s0 + 2 >= n_stripes)
            def _():
                drain_out(0)

            @pl.when(s1 + 2 >= n_stripes)
            def _():
                drain_out(1)

            return carry

        lax.fori_loop(0, n_stripes // 2, body, 0)

    return k


def kernel(inputs, tables):
    b, l, n_cols = inputs.shape
    _, vocab, d = tables.shape
    idx_flat = inputs.astype(jnp.int32).reshape(b * l * n_cols)
    tab_flat = tables.reshape(n_cols * vocab, d)
    j = jnp.arange(_SR, dtype=jnp.int32)
    col = 2 * (j // 16) + (j % 2)
    perm = ((j % 16) // 2) * _PB + col
    offp = col * vocab
    out2d = _make_sc_gather(_L, 0)(idx_flat, tab_flat, perm, offp)
    o5 = out2d.reshape(_L, _B // 8, _N_COLS // 2, 8, 128)
    o5 = _jlayout.with_layout_constraint(
        o5,
        _jlayout.Layout(major_to_minor=(0, 1, 2, 3, 4), tiling=((8, 128),)),
    )
    o5 = o5.transpose(1, 3, 0, 2, 4)
    o5 = _jlayout.with_layout_constraint(
        o5,
        _jlayout.Layout(major_to_minor=(2, 0, 3, 1, 4), tiling=((8, 128),)),
    )
    return o5.reshape(_B, _L, _N_COLS * _D)


# speedup vs baseline: 17.6677x; 1.0168x over previous
import functools

import jax
import jax.numpy as jnp
from jax import lax
from jax.experimental import pallas as pl
from jax.experimental.pallas import tpu as pltpu
from jax.experimental.pallas import tpu_sc as plsc
from jax.experimental import layout as _jlayout

_N_COLS = 26
_VOCAB = 1000
_D = 64
_L = 20
_B = 1024
_PB = _L * _N_COLS
_SR = 8 * _N_COLS
_GC = _SR // 2
_NS = 4


def _make_sc_gather(lp, l0):
    info = plsc.get_sparse_core_info()
    nc, ns = info.num_cores, info.num_subcores
    nw = nc * ns
    nb_per_w = _B // nw
    nblk = nb_per_w // 8
    n_per_w = nb_per_w * _PB
    n_stripes = nblk * lp
    assert _B % (8 * nw) == 0 and n_stripes % 2 == 0

    mesh = plsc.VectorSubcoreMesh(core_axis_name="c", subcore_axis_name="s")

    @functools.partial(
        pl.kernel,
        mesh=mesh,
        compiler_params=pltpu.CompilerParams(
            use_tc_tiling_on_sc=False, needs_layout_passes=False
        ),
        out_type=jax.ShapeDtypeStruct((lp * _B * _N_COLS, _D), jnp.float32),
        scratch_types=[
            pltpu.VMEM((n_per_w,), jnp.int32),
            pltpu.VMEM((_SR,), jnp.int32),
            pltpu.VMEM((_SR,), jnp.int32),
            pltpu.VMEM((n_per_w,), jnp.int32),
            pltpu.VMEM((_NS, _SR, _D), jnp.float32),
            [pltpu.SemaphoreType.DMA] * _NS,
            [pltpu.SemaphoreType.DMA] * _NS,
        ],
    )
    def k(idx_hbm, tab_hbm, perm_hbm, offp_hbm, out_hbm, idx_v, perm_v,
          offp_v, gidx_v, bufs, gsems, osems):
        wid = lax.axis_index("s") * nc + lax.axis_index("c")
        base = wid * n_per_w
        pltpu.sync_copy(idx_hbm.at[pl.ds(base, n_per_w)], idx_v)
        pltpu.sync_copy(perm_hbm, perm_v)
        pltpu.sync_copy(offp_hbm, offp_v)

        def build_gidx(s, carry):
            sbase = (s // lp) * (8 * _PB) + (l0 + s % lp) * _N_COLS
            for kk in range(_SR // 16):
                sl = pl.ds(kk * 16, 16)
                src = perm_v[sl] + sbase
                vals = plsc.load_gather(idx_v, [src])
                gidx_v[pl.ds(s * _SR + kk * 16, 16)] = vals + offp_v[sl]
            return carry

        lax.fori_loop(0, n_stripes, build_gidx, 0)

        def fire_gathers(s, r):
            for j in range(2):
                pltpu.async_copy(
                    tab_hbm.at[gidx_v.at[pl.ds(s * _SR + j * _GC, _GC)]],
                    bufs.at[r, pl.ds(j * _GC, _GC)],
                    gsems[r],
                )

        def drain_gathers(r):
            for j in range(2):
                pltpu.make_async_copy(
                    tab_hbm.at[pl.ds(0, _GC)],
                    bufs.at[r, pl.ds(j * _GC, _GC)],
                    gsems[r],
                ).wait()

        def fire_out(s, r):
            row0 = ((s % lp) * (_B // 8) + wid * nblk + s // lp) * _SR
            pltpu.async_copy(
                bufs.at[r],
                out_hbm.at[pl.ds(row0, _SR)],
                osems[r],
            )

        def drain_out(r):
            pltpu.make_async_copy(
                bufs.at[r],
                out_hbm.at[pl.ds(0, _SR)],
                osems[r],
            ).wait()

        for r in range(_NS):
            fire_gathers(r, r)

        def body(q, carry):
            for r in range(_NS):
                s = q * _NS + r
                drain_gathers(r)
                fire_out(s, r)
                drain_out(r)

                @pl.when(s + _NS < n_stripes)
                def _():
                    fire_gathers(s + _NS, r)

            return carry

        lax.fori_loop(0, n_stripes // _NS, body, 0)

    return k


def kernel(inputs, tables):
    b, l, n_cols = inputs.shape
    _, vocab, d = tables.shape
    idx_flat = inputs.astype(jnp.int32).reshape(b * l * n_cols)
    tab_flat = tables.reshape(n_cols * vocab, d)
    j = jnp.arange(_SR, dtype=jnp.int32)
    col = 2 * (j // 16) + (j % 2)
    perm = ((j % 16) // 2) * _PB + col
    offp = col * vocab
    out2d = _make_sc_gather(_L, 0)(idx_flat, tab_flat, perm, offp)
    o5 = out2d.reshape(_L, _B // 8, _N_COLS // 2, 8, 128)
    o5 = _jlayout.with_layout_constraint(
        o5,
        _jlayout.Layout(major_to_minor=(0, 1, 2, 3, 4), tiling=((8, 128),)),
    )
    o5 = o5.transpose(1, 3, 0, 2, 4)
    o5 = _jlayout.with_layout_constraint(
        o5,
        _jlayout.Layout(major_to_minor=(2, 0, 3, 1, 4), tiling=((8, 128),)),
    )
    return o5.reshape(_B, _L, _N_COLS * _D)
